# Initial kernel scaffold; baseline (speedup 1.0000x reference)
import jax
import jax.numpy as jnp
from jax.experimental import pallas as pl


def kernel(x, edge_index, edge_type, movie_map, user_map, review_map, W_rel, W_self, b_enc, W_cls, b_cls):
    raise NotImplementedError("write your pallas kernel here")


# speedup vs baseline: 7.3418x; 7.3418x over previous
import functools

import jax
import jax.numpy as jnp
import numpy as np
from jax import lax
from jax.experimental import pallas as pl
from jax.experimental.pallas import tpu as pltpu
from jax.experimental.pallas import tpu_sc as plsc

N = 10000
E = 320000
D = 128
R = 3
C = 8
H = D // 2
SNP = R * N
ZPAD = 512
TH = N + ZPAD
CHUNK = 128
NCHUNK_PER_TILE = 157
EDGES_PER_TILE = CHUNK * NCHUNK_PER_TILE
EP = 16 * EDGES_PER_TILE
NPADE = EP - E
RM = 4096
_SC_PARAMS = pltpu.CompilerParams(use_tc_tiling_on_sc=False)

ROWS_PER_TILE = SNP // 16
CROWS = 1800


def _sc_edge_aggregate(srcp, dstp, typp, xs):
    mesh = plsc.VectorSubcoreMesh(core_axis_name="c", subcore_axis_name="s")

    @functools.partial(
        pl.kernel,
        out_type=(
            jax.ShapeDtypeStruct((2, SNP, H), jnp.float32),
            jax.ShapeDtypeStruct((SNP,), jnp.float32),
        ),
        mesh=mesh,
        compiler_params=_SC_PARAMS,
        scratch_types=[
            pltpu.VMEM_SHARED((SNP, H), jnp.float32),
            pltpu.VMEM_SHARED((SNP,), jnp.float32),
            pltpu.VMEM((CHUNK, H), jnp.float32),
            pltpu.VMEM((5, CHUNK), jnp.int32),
            pltpu.VMEM((CHUNK,), jnp.float32),
        ],
    )
    def k(src_h, dst_h, typ_h, xs_h, s_out, cnt_out,
          s_sp, cnt_sp, rows_v, idxv, onesv):
        cid = lax.axis_index("c")
        tid = lax.axis_index("s")

        zf32 = jnp.zeros((16,), jnp.float32)
        of32 = jnp.ones((16,), jnp.float32)

        def fill_body(i, _):
            for j in range(H // 16):
                rows_v[i, pl.ds(j * 16, 16)] = zf32
            return 0
        lax.fori_loop(0, CHUNK, fill_body, 0)
        for j in range(CHUNK // 16):
            onesv[pl.ds(j * 16, 16)] = of32

        row0 = tid * ROWS_PER_TILE

        def zero_body(kk, _):
            pltpu.sync_copy(rows_v, s_sp.at[pl.ds(row0 + kk * CHUNK, CHUNK)])
            return 0
        lax.fori_loop(0, 14, zero_body, 0)
        pltpu.sync_copy(rows_v.at[pl.ds(0, 83)],
                        s_sp.at[pl.ds(row0 + 14 * CHUNK, 83)])

        c0 = tid * CROWS

        def czero_body(kk, _):
            pltpu.sync_copy(rows_v.at[0], cnt_sp.at[pl.ds(c0 + kk * H, H)])
            return 0
        lax.fori_loop(0, CROWS // H, czero_body, 0)

        @pl.when(tid == 0)
        def _():
            def cz2(kk, _):
                pltpu.sync_copy(rows_v.at[0],
                                cnt_sp.at[pl.ds(16 * CROWS + kk * H, H)])
                return 0
            lax.fori_loop(0, (SNP - 16 * CROWS) // H, cz2, 0)

        plsc.subcore_barrier()

        ebase = tid * EDGES_PER_TILE

        def chunk_body(i, _):
            off = ebase + i * CHUNK
            pltpu.sync_copy(src_h.at[pl.ds(off, CHUNK)], idxv.at[0])
            pltpu.sync_copy(dst_h.at[pl.ds(off, CHUNK)], idxv.at[1])
            pltpu.sync_copy(typ_h.at[pl.ds(off, CHUNK)], idxv.at[2])
            for j in range(CHUNK // 16):
                sl = pl.ds(j * 16, 16)
                idxv[3, sl] = idxv[2, sl] * N + idxv[1, sl]
                idxv[4, sl] = idxv[0, sl] + cid * TH
            pltpu.sync_copy(xs_h.at[idxv.at[4]], rows_v)
            pltpu.sync_copy(rows_v, s_sp.at[idxv.at[3]], add=True)

            @pl.when(cid == 0)
            def _():
                pltpu.sync_copy(onesv, cnt_sp.at[idxv.at[3]], add=True)
            return 0
        lax.fori_loop(0, NCHUNK_PER_TILE, chunk_body, 0)

        plsc.subcore_barrier()

        def wb_body(kk, _):
            r = row0 + kk * CHUNK
            pltpu.sync_copy(s_sp.at[pl.ds(r, CHUNK)],
                            s_out.at[cid, pl.ds(r, CHUNK)])
            return 0
        lax.fori_loop(0, 14, wb_body, 0)
        rtail = row0 + 14 * CHUNK
        pltpu.sync_copy(s_sp.at[pl.ds(rtail, 83)],
                        s_out.at[cid, pl.ds(rtail, 83)])

        @pl.when(cid == 0)
        def _():
            pltpu.sync_copy(cnt_sp.at[pl.ds(c0, CROWS)],
                            cnt_out.at[pl.ds(c0, CROWS)])

            @pl.when(tid == 0)
            def _():
                pltpu.sync_copy(cnt_sp.at[pl.ds(16 * CROWS, SNP - 16 * CROWS)],
                                cnt_out.at[pl.ds(16 * CROWS, SNP - 16 * CROWS)])

    return k(srcp, dstp, typp, xs)


def _tc_dense(x, s_arr, cb, W_rel, W_self, b_enc2, W_cls, b_cls2):
    BN = 2000
    nblk = N // BN

    def body(x_ref, s00, s01, s02, s10, s11, s12, c0, c1, c2,
             wrel_ref, wself_ref, benc_ref, wcls_ref, bcls_ref, out_ref):
        xb = x_ref[...]
        acc = jnp.dot(xb, wself_ref[...], preferred_element_type=jnp.float32)
        s_lo = (s00, s01, s02)
        s_hi = (s10, s11, s12)
        cnts = (c0, c1, c2)
        for r in range(R):
            inv = 1.0 / jnp.maximum(cnts[r][...], 1.0)
            sb = jnp.concatenate([s_lo[r][...][0] * inv,
                                  s_hi[r][...][0] * inv], axis=1)
            acc = acc + jnp.dot(sb, wrel_ref[r],
                                preferred_element_type=jnp.float32)
        h = jnp.maximum(acc + benc_ref[...], 0.0)
        out_ref[...] = (jnp.dot(h, wcls_ref[...],
                                preferred_element_type=jnp.float32)
                        + bcls_ref[...])

    in_specs = [pl.BlockSpec((BN, D), lambda i: (i, 0))]
    for c in range(2):
        for r in range(R):
            in_specs.append(pl.BlockSpec(
                (1, BN, H), lambda i, c=c, r=r: (c, 5 * r + i, 0)))
    for r in range(R):
        in_specs.append(pl.BlockSpec(
            (BN, H), lambda i, r=r: (5 * r + i, 0)))
    in_specs += [
        pl.BlockSpec((R, D, D), lambda i: (0, 0, 0)),
        pl.BlockSpec((D, D), lambda i: (0, 0)),
        pl.BlockSpec((1, D), lambda i: (0, 0)),
        pl.BlockSpec((D, C), lambda i: (0, 0)),
        pl.BlockSpec((1, C), lambda i: (0, 0)),
    ]
    return pl.pallas_call(
        body,
        grid=(nblk,),
        in_specs=in_specs,
        out_specs=pl.BlockSpec((BN, C), lambda i: (i, 0)),
        out_shape=jax.ShapeDtypeStruct((N, C), jnp.float32),
    )(x, s_arr, s_arr, s_arr, s_arr, s_arr, s_arr,
      cb, cb, cb, W_rel, W_self, b_enc2, W_cls, b_cls2)


def _sc_review_gather(logits, rmp):
    mesh = plsc.VectorSubcoreMesh(core_axis_name="c", subcore_axis_name="s")
    per_w = RM // 32

    @functools.partial(
        pl.kernel,
        out_type=jax.ShapeDtypeStruct((RM, C), jnp.float32),
        mesh=mesh,
        compiler_params=_SC_PARAMS,
        scratch_types=[
            pltpu.VMEM((1, per_w), jnp.int32),
            pltpu.VMEM((per_w, C), jnp.float32),
        ],
    )
    def k(lg_h, rm_h, out_h, idxv, rows_v):
        cid = lax.axis_index("c")
        tid = lax.axis_index("s")
        wid = tid * 2 + cid
        base = wid * per_w
        pltpu.sync_copy(rm_h.at[pl.ds(base, per_w)], idxv.at[0])
        pltpu.sync_copy(lg_h.at[idxv.at[0]], rows_v)
        pltpu.sync_copy(rows_v, out_h.at[pl.ds(base, per_w)])

    return k(logits, rmp)


_PAD_DELTA = np.zeros((SNP,), np.float32)
for _k in range(NPADE):
    _PAD_DELTA[(_k % R) * N + _k // R] += 1.0


def kernel(x, edge_index, edge_type, movie_map, user_map, review_map,
           W_rel, W_self, b_enc, W_cls, b_cls):
    src = edge_index[0]
    dst = edge_index[1]
    ar = jnp.arange(NPADE, dtype=jnp.int32)
    srcp = jnp.concatenate([src, N + ar % ZPAD])
    dstp = jnp.concatenate([dst, ar // R])
    typp = jnp.concatenate([edge_type, ar % R])
    zrows = jnp.zeros((ZPAD, H), jnp.float32)
    xs = jnp.concatenate([x[:, :H], zrows, x[:, H:], zrows], axis=0)

    s_arr, cnt = _sc_edge_aggregate(srcp, dstp, typp, xs)

    cnt = cnt - jnp.asarray(_PAD_DELTA)
    cb = jnp.broadcast_to(cnt[:, None], (SNP, H))
    logits = _tc_dense(x, s_arr, cb, W_rel, W_self,
                       b_enc.reshape(1, D), W_cls, b_cls.reshape(1, C))

    rmp = jnp.concatenate(
        [review_map, jnp.arange(RM - 4000, dtype=jnp.int32)])
    out = _sc_review_gather(logits, rmp)
    return out[:4000]


# speedup vs baseline: 11.7516x; 1.6006x over previous
import functools

import jax
import jax.numpy as jnp
import numpy as np
from jax import lax
from jax.experimental import pallas as pl
from jax.experimental.pallas import tpu as pltpu
from jax.experimental.pallas import tpu_sc as plsc

N = 10000
E = 320000
D = 128
R = 3
C = 8
HQ = 32
SNP = R * N
ZPAD = 512
TH = N + ZPAD
CHUNK = 128
BLK = 16
NBLK = 10
EDGES_PER_TILE = CHUNK * BLK * NBLK
EP = 16 * EDGES_PER_TILE
NPADE = EP - E
NB = 4
BE = BLK * CHUNK
RM = 4096
_SC_PARAMS = pltpu.CompilerParams(use_tc_tiling_on_sc=False)

ROWS_PER_TILE = SNP // 16
CROWS = 1800


def _sc_edge_aggregate(srcp, dstp, typp, xs, with_cnt):
    mesh = plsc.VectorSubcoreMesh(core_axis_name="c", subcore_axis_name="s")
    out_type = [jax.ShapeDtypeStruct((2, SNP, HQ), jnp.float32)]
    scratch = [
        pltpu.VMEM_SHARED((SNP, HQ), jnp.float32),
        pltpu.VMEM((NB, CHUNK, HQ), jnp.float32),
        pltpu.VMEM((2, 3, BE), jnp.int32),
        pltpu.VMEM((2, 2, BLK, CHUNK), jnp.int32),
        pltpu.SemaphoreType.DMA,
        pltpu.SemaphoreType.DMA,
        pltpu.SemaphoreType.DMA,
    ]
    if with_cnt:
        out_type.append(jax.ShapeDtypeStruct((SNP,), jnp.float32))
        scratch += [
            pltpu.VMEM_SHARED((SNP,), jnp.float32),
            pltpu.VMEM((CHUNK,), jnp.float32),
            pltpu.VMEM((CROWS,), jnp.float32),
            pltpu.SemaphoreType.DMA,
        ]

    @functools.partial(
        pl.kernel,
        out_type=tuple(out_type) if with_cnt else out_type[0],
        mesh=mesh,
        compiler_params=_SC_PARAMS,
        scratch_types=scratch,
    )
    def k(src_h, dst_h, typ_h, xs_h, s_out, *rest):
        if with_cnt:
            (cnt_out, s_sp, rows_v, idxb, sgb, gsem, ssem, isem,
             cnt_sp, onesv, zcv, csem) = rest
        else:
            s_sp, rows_v, idxb, sgb, gsem, ssem, isem = rest
        cid = lax.axis_index("c")
        tid = lax.axis_index("s")

        zf32 = jnp.zeros((16,), jnp.float32)
        of32 = jnp.ones((16,), jnp.float32)

        def fill_body(i, _):
            for j in range(HQ // 16):
                rows_v[0, i, pl.ds(j * 16, 16)] = zf32
            return 0
        lax.fori_loop(0, CHUNK, fill_body, 0)
        if with_cnt:
            for j in range(CHUNK // 16):
                onesv[pl.ds(j * 16, 16)] = of32

            def zc_body(i, _):
                zcv[pl.ds(i * 16, 16)] = zf32
                return 0
            lax.fori_loop(0, CROWS // 16, zc_body, 0)
            zcv[pl.ds(CROWS - 16, 16)] = zf32

        row0 = tid * ROWS_PER_TILE

        def zero_body(kk, _):
            pltpu.sync_copy(rows_v.at[0],
                            s_sp.at[pl.ds(row0 + kk * CHUNK, CHUNK)])
            return 0
        lax.fori_loop(0, 14, zero_body, 0)
        pltpu.sync_copy(rows_v.at[0, pl.ds(0, 83)],
                        s_sp.at[pl.ds(row0 + 14 * CHUNK, 83)])

        if with_cnt:
            c0 = tid * CROWS
            pltpu.sync_copy(zcv, cnt_sp.at[pl.ds(c0, CROWS)])

            @pl.when(tid == 0)
            def _():
                pltpu.sync_copy(zcv.at[pl.ds(0, SNP - 16 * CROWS)],
                                cnt_sp.at[pl.ds(16 * CROWS, SNP - 16 * CROWS)])

        plsc.subcore_barrier()

        ebase = tid * EDGES_PER_TILE
        cth = cid * TH

        def load_idx(g, buf):
            off = ebase + g * BE
            return [pltpu.async_copy(src_h.at[pl.ds(off, BE)],
                                     idxb.at[buf, 0], isem),
                    pltpu.async_copy(dst_h.at[pl.ds(off, BE)],
                                     idxb.at[buf, 1], isem),
                    pltpu.async_copy(typ_h.at[pl.ds(off, BE)],
                                     idxb.at[buf, 2], isem)]

        def compute_sg(buf):
            for b in range(BLK):
                for jq in range(CHUNK // 16):
                    sl = pl.ds(b * CHUNK + jq * 16, 16)
                    col = pl.ds(jq * 16, 16)
                    sgb[buf, 0, b, col] = (
                        idxb[buf, 2, sl] * N + idxb[buf, 1, sl])
                    sgb[buf, 1, b, col] = idxb[buf, 0, sl] + cth

        for d in load_idx(0, 0):
            d.wait()
        compute_sg(0)

        def block_body(g, _):
            buf = g % 2
            nxt = (g + 1) % 2
            idescs = load_idx(lax.rem(g + 1, NBLK), nxt)

            gd = [None] * BLK
            sd = [None] * BLK
            cd = [None] * BLK
            gd[0] = pltpu.async_copy(xs_h.at[sgb.at[buf, 1, 0]],
                                     rows_v.at[0], gsem)
            gd[1] = pltpu.async_copy(xs_h.at[sgb.at[buf, 1, 1]],
                                     rows_v.at[1], gsem)
            for b in range(BLK):
                gd[b].wait()
                sd[b] = pltpu.async_copy(rows_v.at[b % NB],
                                         s_sp.at[sgb.at[buf, 0, b]],
                                         ssem, add=True)
                if with_cnt:
                    cd[b] = pltpu.async_copy(onesv,
                                             cnt_sp.at[sgb.at[buf, 0, b]],
                                             csem, add=True)
                if b >= 2:
                    sd[b - 2].wait()
                    if with_cnt:
                        cd[b - 2].wait()
                if b + 2 < BLK:
                    gd[b + 2] = pltpu.async_copy(
                        xs_h.at[sgb.at[buf, 1, b + 2]],
                        rows_v.at[(b + 2) % NB], gsem)
            for b in (BLK - 2, BLK - 1):
                sd[b].wait()
                if with_cnt:
                    cd[b].wait()

            for d in idescs:
                d.wait()
            compute_sg(nxt)
            return 0
        lax.fori_loop(0, NBLK, block_body, 0)

        plsc.subcore_barrier()

        def wb_body(kk, _):
            r = row0 + kk * CHUNK
            pltpu.sync_copy(s_sp.at[pl.ds(r, CHUNK)],
                            s_out.at[cid, pl.ds(r, CHUNK)])
            return 0
        lax.fori_loop(0, 14, wb_body, 0)
        rtail = row0 + 14 * CHUNK
        pltpu.sync_copy(s_sp.at[pl.ds(rtail, 83)],
                        s_out.at[cid, pl.ds(rtail, 83)])

        if with_cnt:
            @pl.when(cid == 0)
            def _():
                pltpu.sync_copy(cnt_sp.at[pl.ds(c0, CROWS)],
                                cnt_out.at[pl.ds(c0, CROWS)])

                @pl.when(tid == 0)
                def _():
                    pltpu.sync_copy(
                        cnt_sp.at[pl.ds(16 * CROWS, SNP - 16 * CROWS)],
                        cnt_out.at[pl.ds(16 * CROWS, SNP - 16 * CROWS)])

    return k(srcp, dstp, typp, xs)


def _tc_dense(x, s0, s1, cb, W_rel, W_self, b_enc2, W_cls, b_cls2):
    BN = 2000
    nblk = N // BN

    def body(x_ref, q00, q01, q02, q10, q11, q12, q20, q21, q22,
             q30, q31, q32, c0, c1, c2,
             wrel_ref, wself_ref, benc_ref, wcls_ref, bcls_ref, out_ref):
        xb = x_ref[...]
        acc = jnp.dot(xb, wself_ref[...], preferred_element_type=jnp.float32)
        quarters = ((q00, q01, q02), (q10, q11, q12),
                    (q20, q21, q22), (q30, q31, q32))
        cnts = (c0, c1, c2)
        for r in range(R):
            inv = 1.0 / jnp.maximum(cnts[r][...], 1.0)
            sb = jnp.concatenate([quarters[q][r][...][0] * inv
                                  for q in range(4)], axis=1)
            acc = acc + jnp.dot(sb, wrel_ref[r],
                                preferred_element_type=jnp.float32)
        h = jnp.maximum(acc + benc_ref[...], 0.0)
        out_ref[...] = (jnp.dot(h, wcls_ref[...],
                                preferred_element_type=jnp.float32)
                        + bcls_ref[...])

    in_specs = [pl.BlockSpec((BN, D), lambda i: (i, 0))]
    for c in range(2):
        for p in range(2):
            for r in range(R):
                in_specs.append(pl.BlockSpec(
                    (1, BN, HQ), lambda i, c=c, r=r: (c, 5 * r + i, 0)))
    for r in range(R):
        in_specs.append(pl.BlockSpec(
            (BN, HQ), lambda i, r=r: (5 * r + i, 0)))
    in_specs += [
        pl.BlockSpec((R, D, D), lambda i: (0, 0, 0)),
        pl.BlockSpec((D, D), lambda i: (0, 0)),
        pl.BlockSpec((1, D), lambda i: (0, 0)),
        pl.BlockSpec((D, C), lambda i: (0, 0)),
        pl.BlockSpec((1, C), lambda i: (0, 0)),
    ]
    return pl.pallas_call(
        body,
        grid=(nblk,),
        in_specs=in_specs,
        out_specs=pl.BlockSpec((BN, C), lambda i: (i, 0)),
        out_shape=jax.ShapeDtypeStruct((N, C), jnp.float32),
    )(x, s0, s0, s0, s1, s1, s1, s0, s0, s0, s1, s1, s1,
      cb, cb, cb, W_rel, W_self, b_enc2, W_cls, b_cls2)


def _sc_review_gather(logits, rmp):
    mesh = plsc.VectorSubcoreMesh(core_axis_name="c", subcore_axis_name="s")
    per_w = RM // 32

    @functools.partial(
        pl.kernel,
        out_type=jax.ShapeDtypeStruct((RM, C), jnp.float32),
        mesh=mesh,
        compiler_params=_SC_PARAMS,
        scratch_types=[
            pltpu.VMEM((1, per_w), jnp.int32),
            pltpu.VMEM((per_w, C), jnp.float32),
        ],
    )
    def k(lg_h, rm_h, out_h, idxv, rows_v):
        cid = lax.axis_index("c")
        tid = lax.axis_index("s")
        wid = tid * 2 + cid
        base = wid * per_w
        pltpu.sync_copy(rm_h.at[pl.ds(base, per_w)], idxv.at[0])
        pltpu.sync_copy(lg_h.at[idxv.at[0]], rows_v)
        pltpu.sync_copy(rows_v, out_h.at[pl.ds(base, per_w)])

    return k(logits, rmp)


_PAD_DELTA = np.zeros((SNP,), np.float32)
for _k in range(NPADE):
    _PAD_DELTA[(_k % R) * N + (_k // R) % N] += 1.0


def kernel(x, edge_index, edge_type, movie_map, user_map, review_map,
           W_rel, W_self, b_enc, W_cls, b_cls):
    src = edge_index[0]
    dst = edge_index[1]
    ar = jnp.arange(NPADE, dtype=jnp.int32)
    srcp = jnp.concatenate([src, N + ar % ZPAD])
    dstp = jnp.concatenate([dst, (ar // R) % N])
    typp = jnp.concatenate([edge_type, ar % R])
    zrows = jnp.zeros((ZPAD, HQ), jnp.float32)
    xs0 = jnp.concatenate([x[:, 0:32], zrows, x[:, 64:96], zrows], axis=0)
    xs1 = jnp.concatenate([x[:, 32:64], zrows, x[:, 96:128], zrows], axis=0)

    s0, cnt = _sc_edge_aggregate(srcp, dstp, typp, xs0, True)
    s1 = _sc_edge_aggregate(srcp, dstp, typp, xs1, False)

    cnt = cnt - jnp.asarray(_PAD_DELTA)
    cb = jnp.broadcast_to(cnt[:, None], (SNP, HQ))
    logits = _tc_dense(x, s0, s1, cb, W_rel, W_self,
                       b_enc.reshape(1, D), W_cls, b_cls.reshape(1, C))

    rmp = jnp.concatenate(
        [review_map, jnp.arange(RM - 4000, dtype=jnp.int32)])
    out = _sc_review_gather(logits, rmp)
    return out[:4000]


# speedup vs baseline: 13.3024x; 1.1320x over previous
import functools

import jax
import jax.numpy as jnp
import numpy as np
from jax import lax
from jax.experimental import pallas as pl
from jax.experimental.pallas import tpu as pltpu
from jax.experimental.pallas import tpu_sc as plsc

N = 10000
E = 320000
D = 128
R = 3
C = 8
HQ = 32
SNP = R * N
ZPAD = 512
TH = N + ZPAD
CHUNK = 128
BLK = 16
NBLK = 10
EDGES_PER_TILE = CHUNK * BLK * NBLK
EP = 16 * EDGES_PER_TILE
NPADE = EP - E
NB = 8
LA = 4
BE = BLK * CHUNK
RM = 4096
_SC_PARAMS = pltpu.CompilerParams(use_tc_tiling_on_sc=False)

ROWS_PER_TILE = SNP // 16
CROWS = 1800


def _sc_edge_aggregate(srcp, dstp, typp, xs, with_cnt):
    mesh = plsc.VectorSubcoreMesh(core_axis_name="c", subcore_axis_name="s")
    out_type = [jax.ShapeDtypeStruct((2, SNP, HQ), jnp.float32)]
    scratch = [
        pltpu.VMEM_SHARED((SNP, HQ), jnp.float32),
        pltpu.VMEM((NB, CHUNK, HQ), jnp.float32),
        pltpu.VMEM((2, 3, BE), jnp.int32),
        pltpu.VMEM((2, 2, BLK, CHUNK), jnp.int32),
        pltpu.SemaphoreType.DMA,
        pltpu.SemaphoreType.DMA,
        pltpu.SemaphoreType.DMA,
    ]
    if with_cnt:
        out_type.append(jax.ShapeDtypeStruct((SNP,), jnp.float32))
        scratch += [
            pltpu.VMEM_SHARED((SNP,), jnp.float32),
            pltpu.VMEM((CHUNK,), jnp.float32),
            pltpu.VMEM((CROWS,), jnp.float32),
            pltpu.SemaphoreType.DMA,
        ]

    @functools.partial(
        pl.kernel,
        out_type=tuple(out_type) if with_cnt else out_type[0],
        mesh=mesh,
        compiler_params=_SC_PARAMS,
        scratch_types=scratch,
    )
    def k(src_h, dst_h, typ_h, xs_h, s_out, *rest):
        if with_cnt:
            (cnt_out, s_sp, rows_v, idxb, sgb, gsem, ssem, isem,
             cnt_sp, onesv, zcv, csem) = rest
        else:
            s_sp, rows_v, idxb, sgb, gsem, ssem, isem = rest
        cid = lax.axis_index("c")
        tid = lax.axis_index("s")

        zf32 = jnp.zeros((16,), jnp.float32)
        of32 = jnp.ones((16,), jnp.float32)

        def fill_body(i, _):
            for j in range(HQ // 16):
                rows_v[0, i, pl.ds(j * 16, 16)] = zf32
            return 0
        lax.fori_loop(0, CHUNK, fill_body, 0)
        if with_cnt:
            for j in range(CHUNK // 16):
                onesv[pl.ds(j * 16, 16)] = of32

            def zc_body(i, _):
                zcv[pl.ds(i * 16, 16)] = zf32
                return 0
            lax.fori_loop(0, CROWS // 16, zc_body, 0)
            zcv[pl.ds(CROWS - 16, 16)] = zf32

        row0 = tid * ROWS_PER_TILE

        def zero_body(kk, _):
            pltpu.sync_copy(rows_v.at[0],
                            s_sp.at[pl.ds(row0 + kk * CHUNK, CHUNK)])
            return 0
        lax.fori_loop(0, 14, zero_body, 0)
        pltpu.sync_copy(rows_v.at[0, pl.ds(0, 83)],
                        s_sp.at[pl.ds(row0 + 14 * CHUNK, 83)])

        if with_cnt:
            c0 = tid * CROWS
            pltpu.sync_copy(zcv, cnt_sp.at[pl.ds(c0, CROWS)])

            @pl.when(tid == 0)
            def _():
                pltpu.sync_copy(zcv.at[pl.ds(0, SNP - 16 * CROWS)],
                                cnt_sp.at[pl.ds(16 * CROWS, SNP - 16 * CROWS)])

        plsc.subcore_barrier()

        ebase = tid * EDGES_PER_TILE
        cth = cid * TH

        def load_idx(g, buf):
            off = ebase + g * BE
            return [pltpu.async_copy(src_h.at[pl.ds(off, BE)],
                                     idxb.at[buf, 0], isem),
                    pltpu.async_copy(dst_h.at[pl.ds(off, BE)],
                                     idxb.at[buf, 1], isem),
                    pltpu.async_copy(typ_h.at[pl.ds(off, BE)],
                                     idxb.at[buf, 2], isem)]

        def compute_sg(buf):
            for b in range(BLK):
                for jq in range(CHUNK // 16):
                    sl = pl.ds(b * CHUNK + jq * 16, 16)
                    col = pl.ds(jq * 16, 16)
                    sgb[buf, 0, b, col] = (
                        idxb[buf, 2, sl] * N + idxb[buf, 1, sl])
                    sgb[buf, 1, b, col] = idxb[buf, 0, sl] + cth

        for d in load_idx(0, 0):
            d.wait()
        compute_sg(0)

        def block_body(g, _):
            buf = g % 2
            nxt = (g + 1) % 2
            idescs = load_idx(lax.rem(g + 1, NBLK), nxt)

            gd = [None] * BLK
            sd = [None] * BLK
            cd = [None] * BLK
            for b0 in range(LA):
                gd[b0] = pltpu.async_copy(xs_h.at[sgb.at[buf, 1, b0]],
                                          rows_v.at[b0], gsem)
            for b in range(BLK):
                gd[b].wait()
                sd[b] = pltpu.async_copy(rows_v.at[b % NB],
                                         s_sp.at[sgb.at[buf, 0, b]],
                                         ssem, add=True)
                if with_cnt:
                    cd[b] = pltpu.async_copy(onesv,
                                             cnt_sp.at[sgb.at[buf, 0, b]],
                                             csem, add=True)
                if b >= LA:
                    sd[b - LA].wait()
                    if with_cnt:
                        cd[b - LA].wait()
                if b + LA < BLK:
                    gd[b + LA] = pltpu.async_copy(
                        xs_h.at[sgb.at[buf, 1, b + LA]],
                        rows_v.at[(b + LA) % NB], gsem)
            for b in range(BLK - LA, BLK):
                sd[b].wait()
                if with_cnt:
                    cd[b].wait()

            for d in idescs:
                d.wait()
            compute_sg(nxt)
            return 0
        lax.fori_loop(0, NBLK, block_body, 0)

        plsc.subcore_barrier()

        def wb_body(kk, _):
            r = row0 + kk * CHUNK
            pltpu.sync_copy(s_sp.at[pl.ds(r, CHUNK)],
                            s_out.at[cid, pl.ds(r, CHUNK)])
            return 0
        lax.fori_loop(0, 14, wb_body, 0)
        rtail = row0 + 14 * CHUNK
        pltpu.sync_copy(s_sp.at[pl.ds(rtail, 83)],
                        s_out.at[cid, pl.ds(rtail, 83)])

        if with_cnt:
            @pl.when(cid == 0)
            def _():
                pltpu.sync_copy(cnt_sp.at[pl.ds(c0, CROWS)],
                                cnt_out.at[pl.ds(c0, CROWS)])

                @pl.when(tid == 0)
                def _():
                    pltpu.sync_copy(
                        cnt_sp.at[pl.ds(16 * CROWS, SNP - 16 * CROWS)],
                        cnt_out.at[pl.ds(16 * CROWS, SNP - 16 * CROWS)])

    return k(srcp, dstp, typp, xs)


def _tc_dense(x, s0, s1, cb, W_rel, W_self, b_enc2, W_cls, b_cls2):
    BN = 2000
    nblk = N // BN

    def body(x_ref, q00, q01, q02, q10, q11, q12, q20, q21, q22,
             q30, q31, q32, c0, c1, c2,
             wrel_ref, wself_ref, benc_ref, wcls_ref, bcls_ref, out_ref):
        xb = x_ref[...]
        acc = jnp.dot(xb, wself_ref[...], preferred_element_type=jnp.float32)
        quarters = ((q00, q01, q02), (q10, q11, q12),
                    (q20, q21, q22), (q30, q31, q32))
        cnts = (c0, c1, c2)
        for r in range(R):
            inv = 1.0 / jnp.maximum(cnts[r][...], 1.0)
            sb = jnp.concatenate([quarters[q][r][...][0] * inv
                                  for q in range(4)], axis=1)
            acc = acc + jnp.dot(sb, wrel_ref[r],
                                preferred_element_type=jnp.float32)
        h = jnp.maximum(acc + benc_ref[...], 0.0)
        out_ref[...] = (jnp.dot(h, wcls_ref[...],
                                preferred_element_type=jnp.float32)
                        + bcls_ref[...])

    in_specs = [pl.BlockSpec((BN, D), lambda i: (i, 0))]
    for c in range(2):
        for p in range(2):
            for r in range(R):
                in_specs.append(pl.BlockSpec(
                    (1, BN, HQ), lambda i, c=c, r=r: (c, 5 * r + i, 0)))
    for r in range(R):
        in_specs.append(pl.BlockSpec(
            (BN, HQ), lambda i, r=r: (5 * r + i, 0)))
    in_specs += [
        pl.BlockSpec((R, D, D), lambda i: (0, 0, 0)),
        pl.BlockSpec((D, D), lambda i: (0, 0)),
        pl.BlockSpec((1, D), lambda i: (0, 0)),
        pl.BlockSpec((D, C), lambda i: (0, 0)),
        pl.BlockSpec((1, C), lambda i: (0, 0)),
    ]
    return pl.pallas_call(
        body,
        grid=(nblk,),
        in_specs=in_specs,
        out_specs=pl.BlockSpec((BN, C), lambda i: (i, 0)),
        out_shape=jax.ShapeDtypeStruct((N, C), jnp.float32),
    )(x, s0, s0, s0, s1, s1, s1, s0, s0, s0, s1, s1, s1,
      cb, cb, cb, W_rel, W_self, b_enc2, W_cls, b_cls2)


def _sc_review_gather(logits, rmp):
    mesh = plsc.VectorSubcoreMesh(core_axis_name="c", subcore_axis_name="s")
    per_w = RM // 32

    @functools.partial(
        pl.kernel,
        out_type=jax.ShapeDtypeStruct((RM, C), jnp.float32),
        mesh=mesh,
        compiler_params=_SC_PARAMS,
        scratch_types=[
            pltpu.VMEM((1, per_w), jnp.int32),
            pltpu.VMEM((per_w, C), jnp.float32),
        ],
    )
    def k(lg_h, rm_h, out_h, idxv, rows_v):
        cid = lax.axis_index("c")
        tid = lax.axis_index("s")
        wid = tid * 2 + cid
        base = wid * per_w
        pltpu.sync_copy(rm_h.at[pl.ds(base, per_w)], idxv.at[0])
        pltpu.sync_copy(lg_h.at[idxv.at[0]], rows_v)
        pltpu.sync_copy(rows_v, out_h.at[pl.ds(base, per_w)])

    return k(logits, rmp)


_PAD_DELTA = np.zeros((SNP,), np.float32)
for _k in range(NPADE):
    _PAD_DELTA[(_k % R) * N + (_k // R) % N] += 1.0


def kernel(x, edge_index, edge_type, movie_map, user_map, review_map,
           W_rel, W_self, b_enc, W_cls, b_cls):
    src = edge_index[0]
    dst = edge_index[1]
    ar = jnp.arange(NPADE, dtype=jnp.int32)
    srcp = jnp.concatenate([src, N + ar % ZPAD])
    dstp = jnp.concatenate([dst, (ar // R) % N])
    typp = jnp.concatenate([edge_type, ar % R])
    zrows = jnp.zeros((ZPAD, HQ), jnp.float32)
    xs0 = jnp.concatenate([x[:, 0:32], zrows, x[:, 64:96], zrows], axis=0)
    xs1 = jnp.concatenate([x[:, 32:64], zrows, x[:, 96:128], zrows], axis=0)

    s0, cnt = _sc_edge_aggregate(srcp, dstp, typp, xs0, True)
    s1 = _sc_edge_aggregate(srcp, dstp, typp, xs1, False)

    cnt = cnt - jnp.asarray(_PAD_DELTA)
    cb = jnp.broadcast_to(cnt[:, None], (SNP, HQ))
    logits = _tc_dense(x, s0, s1, cb, W_rel, W_self,
                       b_enc.reshape(1, D), W_cls, b_cls.reshape(1, C))

    rmp = jnp.concatenate(
        [review_map, jnp.arange(RM - 4000, dtype=jnp.int32)])
    out = _sc_review_gather(logits, rmp)
    return out[:4000]
